# baseline (speedup 1.0000x reference)
import functools

import jax
import jax.numpy as jnp
from jax.experimental import pallas as pl
from jax.experimental.pallas import tpu as pltpu


def _chamfer_body(q_ref, rt_ref, out_ref, min_ref, *, nq, nr, inv_qk):
    qi = pl.program_id(0)
    ri = pl.program_id(1)
    q = q_ref[...]
    rt = rt_ref[...]
    dots = jnp.dot(
        (-2.0 * q).astype(jnp.bfloat16),
        rt.astype(jnp.bfloat16),
        preferred_element_type=jnp.float32,
    )
    r2 = jnp.sum(rt * rt, axis=0)
    part = r2[None, :] + dots

    @pl.when(ri == 0)
    def _init():
        min_ref[...] = part

    @pl.when(ri != 0)
    def _acc():
        min_ref[...] = jnp.minimum(min_ref[...], part)

    @pl.when(ri == nr - 1)
    def _final():
        q2 = jnp.sum(q * q, axis=1, keepdims=True)
        row_min = jnp.min(min_ref[...], axis=1, keepdims=True)
        partial = jnp.sum(row_min + q2, axis=(0, 1), keepdims=True) * inv_qk

        @pl.when(qi == 0)
        def _first():
            out_ref[...] = partial

        @pl.when(qi != 0)
        def _rest():
            out_ref[...] += partial


def kernel(query, ref):
    q_n, d = query.shape
    r_n, _ = ref.shape
    bq = min(512, q_n)
    br = min(2048, r_n)
    nq, nr = q_n // bq, r_n // br

    body = functools.partial(
        _chamfer_body, nq=nq, nr=nr, inv_qk=1.0 / float(q_n)
    )
    out = pl.pallas_call(
        body,
        grid=(nq, nr),
        in_specs=[
            pl.BlockSpec((bq, d), lambda qi, ri: (qi, 0)),
            pl.BlockSpec((d, br), lambda qi, ri: (0, ri)),
        ],
        out_specs=pl.BlockSpec((1, 1), lambda qi, ri: (0, 0)),
        out_shape=jax.ShapeDtypeStruct((1, 1), jnp.float32),
        scratch_shapes=[pltpu.VMEM((bq, br), jnp.float32)],
    )(query, ref.T)
    return out[0, 0]


# speedup vs baseline: 1.1192x; 1.1192x over previous
import functools

import jax
import jax.numpy as jnp
from jax.experimental import pallas as pl
from jax.experimental.pallas import tpu as pltpu


def _chamfer_body(q_ref, rt_ref, out_ref, min_ref, *, nq, nr, inv_qk):
    qi = pl.program_id(0)
    ri = pl.program_id(1)
    q = q_ref[...]
    rt = rt_ref[...]
    dots = jnp.dot(
        (-2.0 * q).astype(jnp.bfloat16),
        rt.astype(jnp.bfloat16),
        preferred_element_type=jnp.float32,
    )
    r2 = jnp.sum(rt * rt, axis=0)
    m = jnp.min(r2[None, :] + dots, axis=1, keepdims=True)

    @pl.when(ri == 0)
    def _init():
        min_ref[...] = m

    @pl.when(ri != 0)
    def _acc():
        min_ref[...] = jnp.minimum(min_ref[...], m)

    @pl.when(ri == nr - 1)
    def _final():
        q2 = jnp.sum(q * q, axis=1, keepdims=True)
        partial = jnp.sum(min_ref[...] + q2, axis=(0, 1), keepdims=True) * inv_qk

        @pl.when(qi == 0)
        def _first():
            out_ref[...] = partial

        @pl.when(qi != 0)
        def _rest():
            out_ref[...] += partial


def kernel(query, ref):
    q_n, d = query.shape
    r_n, _ = ref.shape
    bq = min(512, q_n)
    br = min(2048, r_n)
    nq, nr = q_n // bq, r_n // br

    body = functools.partial(
        _chamfer_body, nq=nq, nr=nr, inv_qk=1.0 / float(q_n)
    )
    out = pl.pallas_call(
        body,
        grid=(nq, nr),
        in_specs=[
            pl.BlockSpec((bq, d), lambda qi, ri: (qi, 0)),
            pl.BlockSpec((d, br), lambda qi, ri: (0, ri)),
        ],
        out_specs=pl.BlockSpec((1, 1), lambda qi, ri: (0, 0)),
        out_shape=jax.ShapeDtypeStruct((1, 1), jnp.float32),
        scratch_shapes=[pltpu.VMEM((bq, 1), jnp.float32)],
    )(query, ref.T)
    return out[0, 0]


# speedup vs baseline: 1.4390x; 1.2858x over previous
import functools

import jax
import jax.numpy as jnp
from jax.experimental import pallas as pl
from jax.experimental.pallas import tpu as pltpu


def _chamfer_body(q_ref, rt_ref, out_ref, min_ref, r2_ref, *, nq, nr, inv_qk):
    qi = pl.program_id(0)
    ri = pl.program_id(1)
    q = q_ref[...]
    rt = rt_ref[...]

    @pl.when(qi == 0)
    def _r2():
        rtf = rt.astype(jnp.float32)
        r2_ref[pl.ds(ri, 1), :] = jnp.sum(rtf * rtf, axis=0, keepdims=True)

    dots = jnp.dot(
        (-2.0 * q).astype(jnp.bfloat16), rt, preferred_element_type=jnp.float32
    )
    m = jnp.min(r2_ref[pl.ds(ri, 1), :] + dots, axis=1, keepdims=True)

    @pl.when(ri == 0)
    def _init():
        min_ref[...] = m

    @pl.when(ri != 0)
    def _acc():
        min_ref[...] = jnp.minimum(min_ref[...], m)

    @pl.when(ri == nr - 1)
    def _final():
        q2 = jnp.sum(q * q, axis=1, keepdims=True)
        partial = jnp.sum(min_ref[...] + q2, axis=(0, 1), keepdims=True) * inv_qk

        @pl.when(qi == 0)
        def _first():
            out_ref[...] = partial

        @pl.when(qi != 0)
        def _rest():
            out_ref[...] += partial


def kernel(query, ref):
    q_n, d = query.shape
    r_n, _ = ref.shape
    bq = min(512, q_n)
    br = min(2048, r_n)
    nq, nr = q_n // bq, r_n // br

    body = functools.partial(
        _chamfer_body, nq=nq, nr=nr, inv_qk=1.0 / float(q_n)
    )
    out = pl.pallas_call(
        body,
        grid=(nq, nr),
        in_specs=[
            pl.BlockSpec((bq, d), lambda qi, ri: (qi, 0)),
            pl.BlockSpec((d, br), lambda qi, ri: (0, ri)),
        ],
        out_specs=pl.BlockSpec((1, 1), lambda qi, ri: (0, 0)),
        out_shape=jax.ShapeDtypeStruct((1, 1), jnp.float32),
        scratch_shapes=[
            pltpu.VMEM((bq, 1), jnp.float32),
            pltpu.VMEM((nr, br), jnp.float32),
        ],
    )(query, ref.T.astype(jnp.bfloat16))
    return out[0, 0]


# speedup vs baseline: 1.8259x; 1.2689x over previous
import functools

import jax
import jax.numpy as jnp
from jax.experimental import pallas as pl
from jax.experimental.pallas import tpu as pltpu


def _chamfer_body(q_ref, rt_ref, out_ref, min_ref, r2_ref, *, nq, nr, inv_qk):
    qi = pl.program_id(0)
    ri = pl.program_id(1)
    q = q_ref[...]
    rt = rt_ref[...]

    @pl.when(qi == 0)
    def _r2():
        rtf = rt.astype(jnp.float32)
        r2_ref[pl.ds(ri, 1), :] = jnp.sum(rtf * rtf, axis=0, keepdims=True)

    dots = jnp.dot(
        (-2.0 * q).astype(jnp.bfloat16), rt, preferred_element_type=jnp.float32
    )
    m = jnp.min(r2_ref[pl.ds(ri, 1), :] + dots, axis=1, keepdims=True)

    @pl.when(ri == 0)
    def _init():
        min_ref[...] = m

    @pl.when(ri != 0)
    def _acc():
        min_ref[...] = jnp.minimum(min_ref[...], m)

    @pl.when(ri == nr - 1)
    def _final():
        q2 = jnp.sum(q * q, axis=1, keepdims=True)
        partial = jnp.sum(min_ref[...] + q2, axis=(0, 1), keepdims=True) * inv_qk

        @pl.when(qi == 0)
        def _first():
            out_ref[...] = partial

        @pl.when(qi != 0)
        def _rest():
            out_ref[...] += partial


def kernel(query, ref):
    q_n, d = query.shape
    r_n, _ = ref.shape
    bq = min(1024, q_n)
    br = min(2048, r_n)
    nq, nr = q_n // bq, r_n // br

    body = functools.partial(
        _chamfer_body, nq=nq, nr=nr, inv_qk=1.0 / float(q_n)
    )
    out = pl.pallas_call(
        body,
        grid=(nq, nr),
        in_specs=[
            pl.BlockSpec((bq, d), lambda qi, ri: (qi, 0)),
            pl.BlockSpec((d, br), lambda qi, ri: (0, ri)),
        ],
        out_specs=pl.BlockSpec((1, 1), lambda qi, ri: (0, 0)),
        out_shape=jax.ShapeDtypeStruct((1, 1), jnp.float32),
        scratch_shapes=[
            pltpu.VMEM((bq, 1), jnp.float32),
            pltpu.VMEM((nr, br), jnp.float32),
        ],
    )(query, ref.T.astype(jnp.bfloat16))
    return out[0, 0]


# speedup vs baseline: 2.0036x; 1.0973x over previous
import functools

import jax
import jax.numpy as jnp
from jax.experimental import pallas as pl
from jax.experimental.pallas import tpu as pltpu


def _chamfer_body(q_ref, rt_ref, out_ref, min_ref, r2_ref, *, nq, nr, inv_qk):
    qi = pl.program_id(0)
    ri = pl.program_id(1)
    q = q_ref[...]
    rt = rt_ref[...]

    @pl.when(qi == 0)
    def _r2():
        rtf = rt.astype(jnp.float32)
        r2_ref[pl.ds(ri, 1), :] = jnp.sum(rtf * rtf, axis=0, keepdims=True)

    dots = jnp.dot(
        (-2.0 * q).astype(jnp.bfloat16), rt, preferred_element_type=jnp.float32
    )
    m = jnp.min(r2_ref[pl.ds(ri, 1), :] + dots, axis=1, keepdims=True)

    @pl.when(ri == 0)
    def _init():
        min_ref[...] = m

    @pl.when(ri != 0)
    def _acc():
        min_ref[...] = jnp.minimum(min_ref[...], m)

    @pl.when(ri == nr - 1)
    def _final():
        q2 = jnp.sum(q * q, axis=1, keepdims=True)
        partial = jnp.sum(min_ref[...] + q2, axis=(0, 1), keepdims=True) * inv_qk

        @pl.when(qi == 0)
        def _first():
            out_ref[...] = partial

        @pl.when(qi != 0)
        def _rest():
            out_ref[...] += partial


def kernel(query, ref):
    q_n, d = query.shape
    r_n, _ = ref.shape
    bq = min(2048, q_n)
    br = min(2048, r_n)
    nq, nr = q_n // bq, r_n // br

    body = functools.partial(
        _chamfer_body, nq=nq, nr=nr, inv_qk=1.0 / float(q_n)
    )
    out = pl.pallas_call(
        body,
        grid=(nq, nr),
        in_specs=[
            pl.BlockSpec((bq, d), lambda qi, ri: (qi, 0)),
            pl.BlockSpec((d, br), lambda qi, ri: (0, ri)),
        ],
        out_specs=pl.BlockSpec((1, 1), lambda qi, ri: (0, 0)),
        out_shape=jax.ShapeDtypeStruct((1, 1), jnp.float32),
        scratch_shapes=[
            pltpu.VMEM((bq, 1), jnp.float32),
            pltpu.VMEM((nr, br), jnp.float32),
        ],
    )(query, ref.T.astype(jnp.bfloat16))
    return out[0, 0]


# speedup vs baseline: 2.1795x; 1.0878x over previous
import functools

import jax
import jax.numpy as jnp
from jax.experimental import pallas as pl
from jax.experimental.pallas import tpu as pltpu


def _chamfer_body(q_ref, rt_ref, out_ref, min_ref, r2_ref, *, nq, nr, inv_qk):
    qi = pl.program_id(0)
    ri = pl.program_id(1)
    q = q_ref[...]
    rt = rt_ref[...]

    @pl.when(qi == 0)
    def _r2():
        rtf = rt.astype(jnp.float32)
        r2_ref[pl.ds(ri, 1), :] = jnp.sum(rtf * rtf, axis=0, keepdims=True)

    dots = jnp.dot(
        (-2.0 * q).astype(jnp.bfloat16), rt, preferred_element_type=jnp.float32
    )
    m = jnp.min(r2_ref[pl.ds(ri, 1), :] + dots, axis=1, keepdims=True)

    @pl.when(ri == 0)
    def _init():
        min_ref[...] = m

    @pl.when(ri != 0)
    def _acc():
        min_ref[...] = jnp.minimum(min_ref[...], m)

    @pl.when(ri == nr - 1)
    def _final():
        q2 = jnp.sum(q * q, axis=1, keepdims=True)
        partial = jnp.sum(min_ref[...] + q2, axis=(0, 1), keepdims=True) * inv_qk

        @pl.when(qi == 0)
        def _first():
            out_ref[...] = partial

        @pl.when(qi != 0)
        def _rest():
            out_ref[...] += partial


def kernel(query, ref):
    q_n, d = query.shape
    r_n, _ = ref.shape
    bq = min(4096, q_n)
    br = min(2048, r_n)
    nq, nr = q_n // bq, r_n // br

    body = functools.partial(
        _chamfer_body, nq=nq, nr=nr, inv_qk=1.0 / float(q_n)
    )
    out = pl.pallas_call(
        body,
        grid=(nq, nr),
        in_specs=[
            pl.BlockSpec((bq, d), lambda qi, ri: (qi, 0)),
            pl.BlockSpec((d, br), lambda qi, ri: (0, ri)),
        ],
        out_specs=pl.BlockSpec((1, 1), lambda qi, ri: (0, 0)),
        out_shape=jax.ShapeDtypeStruct((1, 1), jnp.float32),
        scratch_shapes=[
            pltpu.VMEM((bq, 1), jnp.float32),
            pltpu.VMEM((nr, br), jnp.float32),
        ],
    )(query, ref.T.astype(jnp.bfloat16))
    return out[0, 0]


# speedup vs baseline: 2.2653x; 1.0393x over previous
import functools

import jax
import jax.numpy as jnp
from jax.experimental import pallas as pl
from jax.experimental.pallas import tpu as pltpu


def _chamfer_body(qt_ref, rb_ref, out_ref, qtb_ref, q2_ref, min_ref, *, nr, inv_q):
    ri = pl.program_id(0)

    @pl.when(ri == 0)
    def _prep_q():
        qt = qt_ref[...]
        qtb_ref[...] = (-2.0 * qt).astype(jnp.bfloat16)
        q2_ref[...] = jnp.sum(qt * qt, axis=0, keepdims=True)

    rb = rb_ref[...]
    dots = jnp.dot(rb, qtb_ref[...], preferred_element_type=jnp.float32)
    rbf = rb.astype(jnp.float32)
    r2 = jnp.sum(rbf * rbf, axis=1, keepdims=True)
    m = jnp.min(r2 + dots, axis=0, keepdims=True)

    @pl.when(ri == 0)
    def _init():
        min_ref[...] = m

    @pl.when(ri != 0)
    def _acc():
        min_ref[...] = jnp.minimum(min_ref[...], m)

    @pl.when(ri == nr - 1)
    def _final():
        out_ref[...] = (
            jnp.sum(min_ref[...] + q2_ref[...], axis=(0, 1), keepdims=True)
            * inv_q
        )


def kernel(query, ref):
    q_n, d = query.shape
    r_n, _ = ref.shape
    br = min(2048, r_n)
    nr = r_n // br

    body = functools.partial(_chamfer_body, nr=nr, inv_q=1.0 / float(q_n))
    out = pl.pallas_call(
        body,
        grid=(nr,),
        in_specs=[
            pl.BlockSpec((d, q_n), lambda ri: (0, 0)),
            pl.BlockSpec((br, d), lambda ri: (ri, 0)),
        ],
        out_specs=pl.BlockSpec((1, 1), lambda ri: (0, 0)),
        out_shape=jax.ShapeDtypeStruct((1, 1), jnp.float32),
        scratch_shapes=[
            pltpu.VMEM((d, q_n), jnp.bfloat16),
            pltpu.VMEM((1, q_n), jnp.float32),
            pltpu.VMEM((1, q_n), jnp.float32),
        ],
    )(query.T, ref.astype(jnp.bfloat16))
    return out[0, 0]


# speedup vs baseline: 2.5240x; 1.1142x over previous
import functools

import jax
import jax.numpy as jnp
from jax.experimental import pallas as pl
from jax.experimental.pallas import tpu as pltpu


def _chamfer_body(qt_ref, rb_ref, out_ref, qtb_ref, q2_ref, min_ref, *, nr, inv_q):
    ri = pl.program_id(0)

    @pl.when(ri == 0)
    def _prep_q():
        qt = qt_ref[...]
        qtb_ref[...] = (-2.0 * qt).astype(jnp.bfloat16)
        q2_ref[...] = jnp.sum(qt * qt, axis=0, keepdims=True)

    rb = rb_ref[...]
    dots = jnp.dot(
        rb.astype(jnp.bfloat16), qtb_ref[...], preferred_element_type=jnp.float32
    )
    r2 = jnp.sum(rb * rb, axis=1, keepdims=True)
    m = jnp.min(r2 + dots, axis=0, keepdims=True)

    @pl.when(ri == 0)
    def _init():
        min_ref[...] = m

    @pl.when(ri != 0)
    def _acc():
        min_ref[...] = jnp.minimum(min_ref[...], m)

    @pl.when(ri == nr - 1)
    def _final():
        out_ref[...] = (
            jnp.sum(min_ref[...] + q2_ref[...], axis=(0, 1), keepdims=True)
            * inv_q
        )


def kernel(query, ref):
    q_n, d = query.shape
    r_n, _ = ref.shape
    br = min(2048, r_n)
    nr = r_n // br

    body = functools.partial(_chamfer_body, nr=nr, inv_q=1.0 / float(q_n))
    out = pl.pallas_call(
        body,
        grid=(nr,),
        in_specs=[
            pl.BlockSpec((d, q_n), lambda ri: (0, 0)),
            pl.BlockSpec((br, d), lambda ri: (ri, 0)),
        ],
        out_specs=pl.BlockSpec((1, 1), lambda ri: (0, 0)),
        out_shape=jax.ShapeDtypeStruct((1, 1), jnp.float32),
        scratch_shapes=[
            pltpu.VMEM((d, q_n), jnp.bfloat16),
            pltpu.VMEM((1, q_n), jnp.float32),
            pltpu.VMEM((1, q_n), jnp.float32),
        ],
    )(query.T, ref)
    return out[0, 0]


# speedup vs baseline: 2.5615x; 1.0148x over previous
import functools

import jax
import jax.numpy as jnp
from jax.experimental import pallas as pl
from jax.experimental.pallas import tpu as pltpu


def _chamfer_body(qt_ref, rb_ref, out_ref, qtb_ref, q2_ref, min_ref, *, nr, inv_q):
    ri = pl.program_id(0)

    @pl.when(ri == 0)
    def _prep_q():
        qt = qt_ref[...]
        qtb_ref[...] = (-2.0 * qt).astype(jnp.bfloat16)
        q2_ref[...] = jnp.sum(qt * qt, axis=0, keepdims=True)

    rb = rb_ref[...]
    dots = jnp.dot(
        rb.astype(jnp.bfloat16), qtb_ref[...], preferred_element_type=jnp.float32
    )
    r2 = jnp.sum(rb * rb, axis=1, keepdims=True)
    m = jnp.min(r2 + dots, axis=0, keepdims=True)

    @pl.when(ri == 0)
    def _init():
        min_ref[...] = m

    @pl.when(ri != 0)
    def _acc():
        min_ref[...] = jnp.minimum(min_ref[...], m)

    @pl.when(ri == nr - 1)
    def _final():
        out_ref[...] = (
            jnp.sum(min_ref[...] + q2_ref[...], axis=(0, 1), keepdims=True)
            * inv_q
        )


def kernel(query, ref):
    q_n, d = query.shape
    r_n, _ = ref.shape
    br = min(4096, r_n)
    nr = r_n // br

    body = functools.partial(_chamfer_body, nr=nr, inv_q=1.0 / float(q_n))
    out = pl.pallas_call(
        body,
        grid=(nr,),
        in_specs=[
            pl.BlockSpec((d, q_n), lambda ri: (0, 0)),
            pl.BlockSpec((br, d), lambda ri: (ri, 0)),
        ],
        out_specs=pl.BlockSpec((1, 1), lambda ri: (0, 0)),
        out_shape=jax.ShapeDtypeStruct((1, 1), jnp.float32),
        scratch_shapes=[
            pltpu.VMEM((d, q_n), jnp.bfloat16),
            pltpu.VMEM((1, q_n), jnp.float32),
            pltpu.VMEM((1, q_n), jnp.float32),
        ],
    )(query.T, ref)
    return out[0, 0]
